# baseline (speedup 1.0000x reference)
import functools

import jax
import jax.numpy as jnp
from jax import lax
from jax.experimental import pallas as pl
from jax.experimental.pallas import tpu as pltpu
from jax.experimental.pallas import tpu_sc as plsc

EX = 256
N_PV = 200
EMBED_DIM = 32
FOURIER = 8
T_OUT = 12
F_OUT = 70

_NC = 2
_NS = 16
_NW = _NC * _NS
_CHUNKS = 16
_CHUNK = 100

_EBLK = 8


def _sc_gather_body(table_hbm, idx_hbm, out_hbm, idx_v, rows_v, sem):
    wid = lax.axis_index("s") * _NC + lax.axis_index("c")
    pltpu.sync_copy(idx_hbm.at[wid], idx_v)
    copies = [
        pltpu.async_copy(table_hbm.at[idx_v.at[j]], rows_v.at[j], sem)
        for j in range(_CHUNKS)
    ]
    for c in copies:
        c.wait()
    pltpu.sync_copy(rows_v, out_hbm.at[wid])


@jax.jit
def _sc_gather(table, idx):
    mesh = plsc.VectorSubcoreMesh(core_axis_name="c", subcore_axis_name="s")
    return pl.kernel(
        _sc_gather_body,
        out_type=jax.ShapeDtypeStruct((_NW, _CHUNKS, _CHUNK, EMBED_DIM), jnp.float32),
        mesh=mesh,
        scratch_types=[
            pltpu.VMEM((_CHUNKS, _CHUNK), jnp.int32),
            pltpu.VMEM((_CHUNKS, _CHUNK, EMBED_DIM), jnp.float32),
            pltpu.SemaphoreType.DMA,
        ],
        compiler_params=pltpu.CompilerParams(use_tc_tiling_on_sc=False),
    )(table, idx)


def _assemble_body(pvt_ref, y_ref, x_ref, tf_ref, emb_ref, az_ref, el_ref, out_ref):
    def clean(v):
        return jnp.where(jnp.isnan(v), jnp.float32(0.0), v)

    tfb = jnp.broadcast_to(tf_ref[:, 0], (_EBLK, N_PV, FOURIER))
    azb = jnp.broadcast_to(az_ref[:, 0], (_EBLK, N_PV, 1))
    elb = jnp.broadcast_to(el_ref[:, 0], (_EBLK, N_PV, 1))
    tile = jnp.concatenate(
        [pvt_ref[...], y_ref[...], x_ref[...], tfb, emb_ref[...], azb, elb],
        axis=-1,
    )
    out_ref[...] = clean(tile).reshape(_EBLK, 1, N_PV, F_OUT)


@jax.jit
def _assemble(pvt, y, x, tf, emb, az, el):
    grid = (EX // _EBLK, T_OUT)
    return pl.pallas_call(
        _assemble_body,
        grid=grid,
        in_specs=[
            pl.BlockSpec((_EBLK, N_PV, 12), lambda i, t: (i, 0, 0)),
            pl.BlockSpec((_EBLK, N_PV, FOURIER), lambda i, t: (i, 0, 0)),
            pl.BlockSpec((_EBLK, N_PV, FOURIER), lambda i, t: (i, 0, 0)),
            pl.BlockSpec((_EBLK, 1, 1, FOURIER), lambda i, t: (i, t, 0, 0)),
            pl.BlockSpec((_EBLK, N_PV, EMBED_DIM), lambda i, t: (i, 0, 0)),
            pl.BlockSpec((_EBLK, 1, 1, 1), lambda i, t: (i, t, 0, 0)),
            pl.BlockSpec((_EBLK, 1, 1, 1), lambda i, t: (i, t, 0, 0)),
        ],
        out_specs=pl.BlockSpec((_EBLK, 1, N_PV, F_OUT), lambda i, t: (i, t, 0, 0)),
        out_shape=jax.ShapeDtypeStruct((EX, T_OUT, N_PV, F_OUT), jnp.float32),
    )(pvt, y, x, tf, emb, az, el)


def kernel(pv_y_osgb_fourier, pv_x_osgb_fourier, pv_system_row_number, pv_x_osgb, pv,
           pv_time_utc_fourier, solar_azimuth, solar_elevation, pv_system_id_embedding):
    idx = pv_system_row_number.astype(jnp.int32).reshape(_NW, _CHUNKS, _CHUNK)
    emb = _sc_gather(pv_system_id_embedding, idx).reshape(EX, N_PV, EMBED_DIM)
    pvt = jnp.transpose(pv[:, :T_OUT], (0, 2, 1))
    tf = pv_time_utc_fourier[:, T_OUT:].reshape(EX, T_OUT, 1, FOURIER)
    az = solar_azimuth[:, T_OUT:].reshape(EX, T_OUT, 1, 1)
    el = solar_elevation[:, T_OUT:].reshape(EX, T_OUT, 1, 1)
    out = _assemble(pvt, pv_y_osgb_fourier, pv_x_osgb_fourier, tf, emb, az, el)
    return out.reshape(EX, T_OUT * N_PV, F_OUT)
